# baseline (speedup 1.0000x reference)
import functools

import jax
import jax.numpy as jnp
from jax import lax
from jax.experimental import pallas as pl
from jax.experimental.pallas import tpu as pltpu
from jax.experimental.pallas import tpu_sc as plsc

_NUM_USERS = 1000000
_DIM = 32
_BATCH = 16384

_NC = 2
_NS = 16
_L = 16
_NW = _NC * _NS
_BPW = _BATCH // _NW
_EPW = _BPW * _DIM
_WAVE = 8
_NWAVE = _BPW // _WAVE

_mesh = plsc.VectorSubcoreMesh(core_axis_name="c", subcore_axis_name="s")


@functools.partial(
    pl.kernel,
    mesh=_mesh,
    out_type=jax.ShapeDtypeStruct((_DIM, _BATCH), jnp.float32),
    scratch_types=[
        pltpu.VMEM((_BPW,), jnp.int32),
        pltpu.VMEM((3, _WAVE, _DIM, 128), jnp.float32),
        pltpu.VMEM((_DIM, _BPW), jnp.float32),
        pltpu.SemaphoreType.DMA,
        pltpu.SemaphoreType.DMA,
        pltpu.SemaphoreType.DMA,
    ],
    compiler_params=pltpu.CompilerParams(needs_layout_passes=False),
)
def _embed_lookup(
    ids_hbm, table_t_hbm, out_hbm, ids_v, blk_v, cols_v, sem0, sem1, sem2
):
    wid = lax.axis_index("s") * _NC + lax.axis_index("c")
    base = wid * _BPW
    sems = (sem0, sem1, sem2)

    pltpu.sync_copy(ids_hbm.at[pl.ds(base, _BPW)], ids_v)

    def load16(m):
        v = ids_v[pl.ds(m * _L, _L)]
        return jnp.minimum(jnp.maximum(v, 0), _NUM_USERS)

    def fire(v16, lane_base, slot):
        for i in range(_WAVE):
            u = v16[lane_base + i]
            ub = pl.multiple_of((u >> 7) << 7, 128)
            for jb in range(_DIM // 8):
                pltpu.make_async_copy(
                    table_t_hbm.at[pl.ds(jb * 8, 8), pl.ds(ub, 128)],
                    blk_v.at[slot, i, pl.ds(jb * 8, 8)],
                    sems[slot],
                ).start()

    def drain_extract(v16, lane_base, slot, w):
        for i in range(_WAVE):
            pltpu.make_async_copy(
                table_t_hbm.at[:, pl.ds(0, 128)],
                blk_v.at[slot, i],
                sems[slot],
            ).wait()
        rows = lax.iota(jnp.int32, _L)
        for i in range(_WAVE):
            ur = jnp.broadcast_to(v16[lane_base + i] & 127, (_L,))
            lo = plsc.load_gather(blk_v.at[slot, i], [rows, ur])
            hi = plsc.load_gather(blk_v.at[slot, i], [rows + _L, ur])
            k = jnp.broadcast_to(w * _WAVE + i, (_L,))
            plsc.store_scatter(cols_v, [rows, k], lo)
            plsc.store_scatter(cols_v, [rows + _L, k], hi)

    def fire_w(w, c):
        fire(load16(w // 2), (c % 2) * _WAVE, c % 3)

    def drain_w(w, c):
        drain_extract(load16(w // 2), (c % 2) * _WAVE, c % 3, w)

    fire_w(0, 0)
    fire_w(1, 1)

    def body(g, _):
        w0 = 6 * g
        for c in range(6):
            fire_w(w0 + c + 2, c + 2)
            drain_w(w0 + c, c)
        return _

    _NGRP = (_NWAVE - 4) // 6
    lax.fori_loop(0, _NGRP, body, 0)

    for w in range(_NWAVE - 4, _NWAVE):
        if w + 2 < _NWAVE:
            fire_w(w + 2, w + 2)
        drain_w(w, w)

    pltpu.sync_copy(cols_v, out_hbm.at[:, pl.ds(base, _BPW)])


def kernel(user_ids, table):
    ids = user_ids.astype(jnp.int32)
    out_t = _embed_lookup(ids, table.T)
    return out_t.T


# speedup vs baseline: 1.0136x; 1.0136x over previous
import functools

import jax
import jax.numpy as jnp
from jax import lax
from jax.experimental import pallas as pl
from jax.experimental.pallas import tpu as pltpu
from jax.experimental.pallas import tpu_sc as plsc

_NUM_USERS = 1000000
_DIM = 32
_BATCH = 16384

_NC = 2
_NS = 16
_L = 16
_NW = _NC * _NS
_BPW = _BATCH // _NW
_EPW = _BPW * _DIM
_WAVE = 8
_NWAVE = _BPW // _WAVE

_mesh = plsc.VectorSubcoreMesh(core_axis_name="c", subcore_axis_name="s")


@functools.partial(
    pl.kernel,
    mesh=_mesh,
    out_type=jax.ShapeDtypeStruct((_DIM, _BATCH), jnp.float32),
    scratch_types=[
        pltpu.VMEM((_BPW,), jnp.int32),
        pltpu.VMEM((3, _WAVE, _DIM, 128), jnp.float32),
        pltpu.VMEM((_DIM, _BPW), jnp.float32),
        pltpu.SemaphoreType.DMA,
        pltpu.SemaphoreType.DMA,
        pltpu.SemaphoreType.DMA,
    ],
    compiler_params=pltpu.CompilerParams(needs_layout_passes=False),
)
def _embed_lookup(
    ids_hbm, table_t_hbm, out_hbm, ids_v, blk_v, cols_v, sem0, sem1, sem2
):
    wid = lax.axis_index("s") * _NC + lax.axis_index("c")
    base = wid * _BPW
    sems = (sem0, sem1, sem2)

    pltpu.sync_copy(ids_hbm.at[pl.ds(base, _BPW)], ids_v)

    def load16(m):
        v = ids_v[pl.ds(m * _L, _L)]
        return jnp.minimum(jnp.maximum(v, 0), _NUM_USERS)

    def fire(v16, lane_base, slot):
        for i in range(_WAVE):
            u = v16[lane_base + i]
            ub = pl.multiple_of((u >> 7) << 7, 128)
            pltpu.make_async_copy(
                table_t_hbm.at[:, pl.ds(ub, 128)],
                blk_v.at[slot, i],
                sems[slot],
            ).start()

    def drain_extract(v16, lane_base, slot, w):
        for i in range(_WAVE):
            pltpu.make_async_copy(
                table_t_hbm.at[:, pl.ds(0, 128)],
                blk_v.at[slot, i],
                sems[slot],
            ).wait()
        rows = lax.iota(jnp.int32, _L)
        for i in range(_WAVE):
            ur = jnp.broadcast_to(v16[lane_base + i] & 127, (_L,))
            lo = plsc.load_gather(blk_v.at[slot, i], [rows, ur])
            hi = plsc.load_gather(blk_v.at[slot, i], [rows + _L, ur])
            k = jnp.broadcast_to(w * _WAVE + i, (_L,))
            plsc.store_scatter(cols_v, [rows, k], lo)
            plsc.store_scatter(cols_v, [rows + _L, k], hi)

    def fire_w(w, c):
        fire(load16(w // 2), (c % 2) * _WAVE, c % 3)

    def drain_w(w, c):
        drain_extract(load16(w // 2), (c % 2) * _WAVE, c % 3, w)

    fire_w(0, 0)
    fire_w(1, 1)

    def body(g, _):
        w0 = 6 * g
        for c in range(6):
            fire_w(w0 + c + 2, c + 2)
            drain_w(w0 + c, c)
        return _

    _NGRP = (_NWAVE - 4) // 6
    lax.fori_loop(0, _NGRP, body, 0)

    for w in range(_NWAVE - 4, _NWAVE):
        if w + 2 < _NWAVE:
            fire_w(w + 2, w + 2)
        drain_w(w, w)

    pltpu.sync_copy(cols_v, out_hbm.at[:, pl.ds(base, _BPW)])


def kernel(user_ids, table):
    ids = user_ids.astype(jnp.int32)
    out_t = _embed_lookup(ids, table.T)
    return out_t.T
